# baseline (speedup 1.0000x reference)
import functools

import jax
import jax.numpy as jnp
from jax.experimental import pallas as pl
from jax.experimental.pallas import tpu as pltpu

B = 4
NP = 200000
PTS_IN = 4
EMB = 80
BEV_C = 128
BEV_H = 128
BEV_W = 128
HW = BEV_H * BEV_W
X_MIN, Y_MIN = -50.0, -50.0
X_MAX, Y_MAX = 50.0, 50.0
MX = (X_MAX - X_MIN) / BEV_W
MY = (Y_MAX - Y_MIN) / BEV_H

CA = 2000
NA = NP // CA
CB = 2048
NB = HW // CB


def _mlp_body(pts_ref, w1_ref, b1_ref, w2_ref, b2_ref, emb_ref, ind_ref):
    x = pts_ref[0]
    h = jnp.maximum(
        jnp.dot(x, w1_ref[...], preferred_element_type=jnp.float32) + b1_ref[...], 0.0)
    e = jnp.maximum(
        jnp.dot(h, w2_ref[...], preferred_element_type=jnp.float32) + b2_ref[...], 0.0)
    emb_ref[0] = e
    xt = x.T
    xx = xt[0:1, :]
    yy = xt[1:2, :]
    ix = jnp.floor((xx - X_MIN) / MX).astype(jnp.int32)
    iy = jnp.floor((yy - Y_MIN) / MY).astype(jnp.int32)
    valid = (ix >= 0) & (ix < BEV_W) & (iy >= 0) & (iy < BEV_H)
    ind = jnp.clip(iy * BEV_W + ix, 0, HW - 1)
    ind_ref[0] = jnp.where(valid, ind, HW)


def _mlp_embed(points, W1, b1, W2, b2):
    grid = (B, NA)
    emb, ind = pl.pallas_call(
        _mlp_body,
        grid=grid,
        in_specs=[
            pl.BlockSpec((1, CA, PTS_IN), lambda b, j: (b, j, 0)),
            pl.BlockSpec((PTS_IN, EMB), lambda b, j: (0, 0)),
            pl.BlockSpec((1, EMB), lambda b, j: (0, 0)),
            pl.BlockSpec((EMB, EMB), lambda b, j: (0, 0)),
            pl.BlockSpec((1, EMB), lambda b, j: (0, 0)),
        ],
        out_specs=[
            pl.BlockSpec((1, CA, EMB), lambda b, j: (b, j, 0)),
            pl.BlockSpec((1, 1, CA), lambda b, j: (b * NA + j, 0, 0)),
        ],
        out_shape=[
            jax.ShapeDtypeStruct((B, NP, EMB), jnp.float32),
            jax.ShapeDtypeStruct((B * NA, 1, CA), jnp.int32),
        ],
    )(points, W1, b1.reshape(1, EMB), W2, b2.reshape(1, EMB))
    return emb, ind.reshape(B, NP)


def _tail1_body(sum_ref, cnt_ref, wpt_ref, bp_ref, acc_ref):
    @pl.when((pl.program_id(0) == 0) & (pl.program_id(1) == 0))
    def _():
        acc_ref[...] = jnp.zeros_like(acc_ref)

    s = sum_ref[0]
    c = cnt_ref[0]
    recip = 1.0 / jnp.maximum(c, 1.0)
    mean = s * recip.T
    conv = jnp.dot(mean, wpt_ref[...],
                   preferred_element_type=jnp.float32) + bp_ref[...]
    acc_ref[0:1, :] += jnp.sum(conv, axis=0, keepdims=True)
    acc_ref[1:2, :] += jnp.sum(conv * conv, axis=0, keepdims=True)


def _tail2_body(sum_ref, cnt_ref, wpt_ref, bp_ref, acc_ref, g_ref, bt_ref, out_ref):
    s = sum_ref[0]
    c = cnt_ref[0]
    recip = 1.0 / jnp.maximum(c, 1.0)
    mean = s * recip.T
    conv = jnp.dot(mean, wpt_ref[...],
                   preferred_element_type=jnp.float32) + bp_ref[...]
    n = float(B * HW)
    m = acc_ref[0:1, :] / n
    v = acc_ref[1:2, :] / n - m * m
    scale = g_ref[...] * jax.lax.rsqrt(v + 1e-5)
    shift = bt_ref[...] - m * scale
    y = jnp.maximum(conv * scale + shift, 0.0)
    out_ref[0] = y.T


def _tail(sums, cnts, Wp, bp, gamma, beta):
    wpt = Wp.T
    bp2 = bp.reshape(1, BEV_C)
    g2 = gamma.reshape(1, BEV_C)
    bt2 = beta.reshape(1, BEV_C)
    grid = (B, NB)
    sum_spec = pl.BlockSpec((1, CB, EMB), lambda b, j: (b, j, 0))
    cnt_spec = pl.BlockSpec((1, 1, CB), lambda b, j: (b * NB + j, 0, 0))
    wpt_spec = pl.BlockSpec((EMB, BEV_C), lambda b, j: (0, 0))
    v_spec = pl.BlockSpec((1, BEV_C), lambda b, j: (0, 0))
    acc = pl.pallas_call(
        _tail1_body,
        grid=grid,
        in_specs=[sum_spec, cnt_spec, wpt_spec, v_spec],
        out_specs=pl.BlockSpec((8, BEV_C), lambda b, j: (0, 0)),
        out_shape=jax.ShapeDtypeStruct((8, BEV_C), jnp.float32),
    )(sums, cnts, wpt, bp2)
    out = pl.pallas_call(
        _tail2_body,
        grid=grid,
        in_specs=[sum_spec, cnt_spec, wpt_spec, v_spec,
                  pl.BlockSpec((8, BEV_C), lambda b, j: (0, 0)),
                  v_spec, v_spec],
        out_specs=pl.BlockSpec((1, BEV_C, CB), lambda b, j: (b, 0, j)),
        out_shape=jax.ShapeDtypeStruct((B, BEV_C, HW), jnp.float32),
    )(sums, cnts, wpt, bp2, acc, g2, bt2)
    return out.reshape(B, BEV_C, BEV_H, BEV_W)


def kernel(points, W1, b1, W2, b2, Wp, bp, gamma, beta):
    emb, ind = _mlp_embed(points, W1, b1, W2, b2)
    g = (jnp.arange(B, dtype=jnp.int32)[:, None] * (HW + 1) + ind).reshape(-1)
    sums = jax.ops.segment_sum(emb.reshape(-1, EMB), g, num_segments=B * (HW + 1))
    cnts = jax.ops.segment_sum(jnp.ones((B * NP,), jnp.float32), g,
                               num_segments=B * (HW + 1))
    sums3 = sums.reshape(B, HW + 1, EMB)[:, :HW]
    cnts3 = cnts.reshape(B, HW + 1)[:, :HW].reshape(B * NB, 1, CB)
    return _tail(sums3, cnts3, Wp, bp, gamma, beta)


# speedup vs baseline: 2.4290x; 2.4290x over previous
import functools

import jax
import jax.numpy as jnp
from jax import lax
from jax.experimental import pallas as pl
from jax.experimental.pallas import tpu as pltpu
from jax.experimental.pallas import tpu_sc as plsc

B = 4
NP = 200000
PTS_IN = 4
EMB = 80
BEV_C = 128
BEV_H = 128
BEV_W = 128
HW = BEV_H * BEV_W
X_MIN, Y_MIN = -50.0, -50.0
X_MAX, Y_MAX = 50.0, 50.0
MX = (X_MAX - X_MIN) / BEV_W
MY = (Y_MAX - Y_MIN) / BEV_H

CA = 2000
NA = NP // CA
CB = 2048
NB = HW // CB


def _mlp_body(pts_ref, w1_ref, b1_ref, w2_ref, b2_ref, emb_ref, ind_ref):
    x = pts_ref[0]
    h = jnp.maximum(
        jnp.dot(x, w1_ref[...], preferred_element_type=jnp.float32) + b1_ref[...], 0.0)
    e = jnp.maximum(
        jnp.dot(h, w2_ref[...], preferred_element_type=jnp.float32) + b2_ref[...], 0.0)
    emb_ref[...] = jnp.concatenate(
        [e, jnp.ones((CA, 1), jnp.float32), jnp.zeros((CA, 127 - EMB), jnp.float32)],
        axis=1)
    xt = x.T
    xx = xt[0:1, :]
    yy = xt[1:2, :]
    ix = jnp.floor((xx - X_MIN) / MX).astype(jnp.int32)
    iy = jnp.floor((yy - Y_MIN) / MY).astype(jnp.int32)
    valid = (ix >= 0) & (ix < BEV_W) & (iy >= 0) & (iy < BEV_H)
    ind = jnp.clip(iy * BEV_W + ix, 0, HW - 1)
    ind_ref[0] = jnp.where(valid, ind, HW)


def _mlp_embed(points, W1, b1, W2, b2):
    grid = (B, NA)
    emb, ind = pl.pallas_call(
        _mlp_body,
        grid=grid,
        in_specs=[
            pl.BlockSpec((1, CA, PTS_IN), lambda b, j: (b, j, 0)),
            pl.BlockSpec((PTS_IN, EMB), lambda b, j: (0, 0)),
            pl.BlockSpec((1, EMB), lambda b, j: (0, 0)),
            pl.BlockSpec((EMB, EMB), lambda b, j: (0, 0)),
            pl.BlockSpec((1, EMB), lambda b, j: (0, 0)),
        ],
        out_specs=[
            pl.BlockSpec((CA, 128), lambda b, j: (b * NA + j, 0)),
            pl.BlockSpec((1, 1, CA), lambda b, j: (b * NA + j, 0, 0)),
        ],
        out_shape=[
            jax.ShapeDtypeStruct((B * NP, 128), jnp.float32),
            jax.ShapeDtypeStruct((B * NA, 1, CA), jnp.int32),
        ],
    )(points, W1, b1.reshape(1, EMB), W2, b2.reshape(1, EMB))
    return emb, ind.reshape(B * NP)


K = 128
NCH = NP // K
TAIL = NP - NCH * K
HALF = HW // 2
NROWS = 8320
ZR = NROWS // 16
OR_ = HALF // 16
OB = 128


def _sc_scatter_body(emb_hbm, ind_hbm, z2_hbm, sums_hbm,
                     acc, eb, ib, ebt, ibt):
    cid = lax.axis_index("c")
    sid = lax.axis_index("s")

    def remap(idx_ref, n, off):
        for i in range(n // 16):
            v = idx_ref[pl.ds(i * 16, 16)] - off
            bad = (v < 0) | (v >= HALF)
            idx_ref[pl.ds(i * 16, 16)] = jnp.where(bad, HALF, v)

    for rep in range(2):
        b = cid * 2 + rep
        for h in range(2):
            off = h * HALF
            pltpu.sync_copy(z2_hbm, eb)
            r0 = sid * ZR
            for j in range(4):
                pltpu.sync_copy(eb, acc.at[pl.ds(r0 + j * OB, OB)])
            pltpu.sync_copy(eb.at[pl.ds(0, 8)], acc.at[pl.ds(r0 + 4 * OB, 8)])
            plsc.subcore_barrier()

            def chunk_body(i, _):
                c = i * 16 + sid

                @pl.when(c < NCH)
                def _():
                    base = b * NP + c * K
                    pltpu.sync_copy(ind_hbm.at[pl.ds(base, K)], ib)
                    pltpu.sync_copy(emb_hbm.at[pl.ds(base, K)], eb)
                    remap(ib, K, off)
                    pltpu.sync_copy(eb, acc.at[ib], add=True)
                return ()

            lax.fori_loop(0, (NCH + 15) // 16, chunk_body, (), unroll=False)

            @pl.when(sid == NCH % 16)
            def _():
                base = b * NP + NCH * K
                pltpu.sync_copy(ind_hbm.at[pl.ds(base, TAIL)], ibt)
                pltpu.sync_copy(emb_hbm.at[pl.ds(base, TAIL)], ebt)
                remap(ibt, TAIL, off)
                pltpu.sync_copy(ebt, acc.at[ibt], add=True)

            plsc.subcore_barrier()

            o0 = sid * OR_
            for j in range(OR_ // OB):
                pltpu.sync_copy(acc.at[pl.ds(o0 + j * OB, OB)], eb)
                pltpu.sync_copy(eb, sums_hbm.at[b, pl.ds(off + o0 + j * OB, OB)])
            plsc.subcore_barrier()


@functools.partial(
    pl.kernel,
    out_type=jax.ShapeDtypeStruct((B, HW, 128), jnp.float32),
    mesh=plsc.VectorSubcoreMesh(core_axis_name="c", subcore_axis_name="s"),
    scratch_types=[
        pltpu.VMEM_SHARED((NROWS, 128), jnp.float32),
        pltpu.VMEM((K, 128), jnp.float32),
        pltpu.VMEM((K,), jnp.int32),
        pltpu.VMEM((TAIL, 128), jnp.float32),
        pltpu.VMEM((TAIL,), jnp.int32),
    ],
)
def _sc_scatter(emb_hbm, ind_hbm, z2_hbm, sums_hbm,
                acc, eb, ib, ebt, ibt):
    _sc_scatter_body(emb_hbm, ind_hbm, z2_hbm, sums_hbm,
                     acc, eb, ib, ebt, ibt)


def _mean_conv(blk, wpt_ref, bp_ref):
    s = blk[:, :EMB]
    c = blk[:, EMB:EMB + 1]
    recip = 1.0 / jnp.maximum(c, 1.0)
    mean = s * recip
    return jnp.dot(mean, wpt_ref[...],
                   preferred_element_type=jnp.float32) + bp_ref[...]


def _tail1_body(sum_ref, wpt_ref, bp_ref, acc_ref):
    @pl.when((pl.program_id(0) == 0) & (pl.program_id(1) == 0))
    def _():
        acc_ref[...] = jnp.zeros_like(acc_ref)

    conv = _mean_conv(sum_ref[0], wpt_ref, bp_ref)
    acc_ref[0:1, :] += jnp.sum(conv, axis=0, keepdims=True)
    acc_ref[1:2, :] += jnp.sum(conv * conv, axis=0, keepdims=True)


def _tail2_body(sum_ref, wpt_ref, bp_ref, acc_ref, g_ref, bt_ref, out_ref):
    conv = _mean_conv(sum_ref[0], wpt_ref, bp_ref)
    n = float(B * HW)
    m = acc_ref[0:1, :] / n
    v = acc_ref[1:2, :] / n - m * m
    scale = g_ref[...] * jax.lax.rsqrt(v + 1e-5)
    shift = bt_ref[...] - m * scale
    y = jnp.maximum(conv * scale + shift, 0.0)
    out_ref[0] = y.T


def _tail(sums, Wp, bp, gamma, beta):
    wpt = Wp.T
    bp2 = bp.reshape(1, BEV_C)
    g2 = gamma.reshape(1, BEV_C)
    bt2 = beta.reshape(1, BEV_C)
    grid = (B, NB)
    sum_spec = pl.BlockSpec((1, CB, 128), lambda b, j: (b, j, 0))
    wpt_spec = pl.BlockSpec((EMB, BEV_C), lambda b, j: (0, 0))
    v_spec = pl.BlockSpec((1, BEV_C), lambda b, j: (0, 0))
    acc = pl.pallas_call(
        _tail1_body,
        grid=grid,
        in_specs=[sum_spec, wpt_spec, v_spec],
        out_specs=pl.BlockSpec((8, BEV_C), lambda b, j: (0, 0)),
        out_shape=jax.ShapeDtypeStruct((8, BEV_C), jnp.float32),
    )(sums, wpt, bp2)
    out = pl.pallas_call(
        _tail2_body,
        grid=grid,
        in_specs=[sum_spec, wpt_spec, v_spec,
                  pl.BlockSpec((8, BEV_C), lambda b, j: (0, 0)),
                  v_spec, v_spec],
        out_specs=pl.BlockSpec((1, BEV_C, CB), lambda b, j: (b, 0, j)),
        out_shape=jax.ShapeDtypeStruct((B, BEV_C, HW), jnp.float32),
    )(sums, wpt, bp2, acc, g2, bt2)
    return out.reshape(B, BEV_C, BEV_H, BEV_W)


def kernel(points, W1, b1, W2, b2, Wp, bp, gamma, beta):
    emb, ind = _mlp_embed(points, W1, b1, W2, b2)
    z2 = jnp.zeros((OB, 128), jnp.float32)
    sums = _sc_scatter(emb, ind, z2)
    return _tail(sums, Wp, bp, gamma, beta)


# speedup vs baseline: 3.0255x; 1.2456x over previous
import functools

import jax
import jax.numpy as jnp
from jax import lax
from jax.experimental import pallas as pl
from jax.experimental.pallas import tpu as pltpu
from jax.experimental.pallas import tpu_sc as plsc

B = 4
NP = 200000
PTS_IN = 4
EMB = 80
BEV_C = 128
BEV_H = 128
BEV_W = 128
HW = BEV_H * BEV_W
X_MIN, Y_MIN = -50.0, -50.0
X_MAX, Y_MAX = 50.0, 50.0
MX = (X_MAX - X_MIN) / BEV_W
MY = (Y_MAX - Y_MIN) / BEV_H

CA = 2000
NA = NP // CA
CB = 2048
NB = HW // CB


def _mlp_body(pts_ref, w1_ref, b1_ref, w2_ref, b2_ref, emb_ref, ind_ref):
    x = pts_ref[0]
    h = jnp.maximum(
        jnp.dot(x, w1_ref[...], preferred_element_type=jnp.float32) + b1_ref[...], 0.0)
    e = jnp.maximum(
        jnp.dot(h, w2_ref[...], preferred_element_type=jnp.float32) + b2_ref[...], 0.0)
    emb_ref[...] = jnp.concatenate(
        [e, jnp.ones((CA, 1), jnp.float32), jnp.zeros((CA, 127 - EMB), jnp.float32)],
        axis=1)
    xt = x.T
    xx = xt[0:1, :]
    yy = xt[1:2, :]
    ix = jnp.floor((xx - X_MIN) / MX).astype(jnp.int32)
    iy = jnp.floor((yy - Y_MIN) / MY).astype(jnp.int32)
    valid = (ix >= 0) & (ix < BEV_W) & (iy >= 0) & (iy < BEV_H)
    ind = jnp.clip(iy * BEV_W + ix, 0, HW - 1)
    ind_ref[0] = jnp.where(valid, ind, HW)


def _mlp_embed(points, W1, b1, W2, b2):
    grid = (B, NA)
    emb, ind = pl.pallas_call(
        _mlp_body,
        grid=grid,
        in_specs=[
            pl.BlockSpec((1, CA, PTS_IN), lambda b, j: (b, j, 0)),
            pl.BlockSpec((PTS_IN, EMB), lambda b, j: (0, 0)),
            pl.BlockSpec((1, EMB), lambda b, j: (0, 0)),
            pl.BlockSpec((EMB, EMB), lambda b, j: (0, 0)),
            pl.BlockSpec((1, EMB), lambda b, j: (0, 0)),
        ],
        out_specs=[
            pl.BlockSpec((CA, 128), lambda b, j: (b * NA + j, 0)),
            pl.BlockSpec((1, 1, CA), lambda b, j: (b * NA + j, 0, 0)),
        ],
        out_shape=[
            jax.ShapeDtypeStruct((B * NP, 128), jnp.float32),
            jax.ShapeDtypeStruct((B * NA, 1, CA), jnp.int32),
        ],
    )(points, W1, b1.reshape(1, EMB), W2, b2.reshape(1, EMB))
    return emb, ind.reshape(B * NP)


K = 128
NCH = NP // K
TAIL = NP - NCH * K
HALF = HW // 2
NROWS = 8320
ZR = NROWS // 16
OR_ = HALF // 16
OB = 128


def _sc_scatter_body(emb_hbm, ind_hbm, z2_hbm, sums_hbm,
                     acc, eb, ib, eb1, ib1, ebt, ibt, sem0, sem1):
    cid = lax.axis_index("c")
    sid = lax.axis_index("s")

    def remap(idx_ref, n, off):
        for i in range(n // 16):
            v = idx_ref[pl.ds(i * 16, 16)] - off
            bad = (v < 0) | (v >= HALF)
            idx_ref[pl.ds(i * 16, 16)] = jnp.where(bad, HALF, v)

    for rep in range(2):
        b = cid * 2 + rep
        for h in range(2):
            off = h * HALF
            pltpu.sync_copy(z2_hbm, eb)
            r0 = sid * ZR
            for j in range(4):
                pltpu.sync_copy(eb, acc.at[pl.ds(r0 + j * OB, OB)])
            pltpu.sync_copy(eb.at[pl.ds(0, 8)], acc.at[pl.ds(r0 + 4 * OB, 8)])
            plsc.subcore_barrier()

            def g_start(i, ebx, ibx, sem):
                c = i * 16 + sid

                @pl.when(c < NCH)
                def _():
                    base = b * NP + c * K
                    pltpu.async_copy(ind_hbm.at[pl.ds(base, K)], ibx, sem)
                    pltpu.async_copy(emb_hbm.at[pl.ds(base, K)], ebx, sem)

            def g_finish(i, ebx, ibx, sem):
                c = i * 16 + sid

                @pl.when(c < NCH)
                def _():
                    base = b * NP + c * K
                    pltpu.make_async_copy(
                        ind_hbm.at[pl.ds(base, K)], ibx, sem).wait()
                    pltpu.make_async_copy(
                        emb_hbm.at[pl.ds(base, K)], ebx, sem).wait()
                    remap(ibx, K, off)
                    pltpu.sync_copy(ebx, acc.at[ibx], add=True)

            def pair_body(t, _):
                g_start(2 * t + 1, eb1, ib1, sem1)
                g_finish(2 * t, eb, ib, sem0)
                g_start(2 * t + 2, eb, ib, sem0)
                g_finish(2 * t + 1, eb1, ib1, sem1)
                return ()

            g_start(0, eb, ib, sem0)
            lax.fori_loop(0, (NCH + 15) // 16 // 2, pair_body, (),
                          unroll=False)

            @pl.when(sid == NCH % 16)
            def _():
                base = b * NP + NCH * K
                pltpu.sync_copy(ind_hbm.at[pl.ds(base, TAIL)], ibt)
                pltpu.sync_copy(emb_hbm.at[pl.ds(base, TAIL)], ebt)
                remap(ibt, TAIL, off)
                pltpu.sync_copy(ebt, acc.at[ibt], add=True)

            plsc.subcore_barrier()

            o0 = sid * OR_
            for j in range(OR_ // OB):
                pltpu.sync_copy(acc.at[pl.ds(o0 + j * OB, OB)], eb)
                pltpu.sync_copy(eb, sums_hbm.at[b, pl.ds(off + o0 + j * OB, OB)])
            plsc.subcore_barrier()


@functools.partial(
    pl.kernel,
    out_type=jax.ShapeDtypeStruct((B, HW, 128), jnp.float32),
    mesh=plsc.VectorSubcoreMesh(core_axis_name="c", subcore_axis_name="s"),
    scratch_types=[
        pltpu.VMEM_SHARED((NROWS, 128), jnp.float32),
        pltpu.VMEM((K, 128), jnp.float32),
        pltpu.VMEM((K,), jnp.int32),
        pltpu.VMEM((K, 128), jnp.float32),
        pltpu.VMEM((K,), jnp.int32),
        pltpu.VMEM((TAIL, 128), jnp.float32),
        pltpu.VMEM((TAIL,), jnp.int32),
        pltpu.SemaphoreType.DMA,
        pltpu.SemaphoreType.DMA,
    ],
)
def _sc_scatter(emb_hbm, ind_hbm, z2_hbm, sums_hbm,
                acc, eb, ib, eb1, ib1, ebt, ibt, sem0, sem1):
    _sc_scatter_body(emb_hbm, ind_hbm, z2_hbm, sums_hbm,
                     acc, eb, ib, eb1, ib1, ebt, ibt, sem0, sem1)


def _mean_conv(blk, wpt_ref, bp_ref):
    s = blk[:, :EMB]
    c = blk[:, EMB:EMB + 1]
    recip = 1.0 / jnp.maximum(c, 1.0)
    mean = s * recip
    return jnp.dot(mean, wpt_ref[...],
                   preferred_element_type=jnp.float32) + bp_ref[...]


def _tail1_body(sum_ref, wpt_ref, bp_ref, acc_ref):
    @pl.when((pl.program_id(0) == 0) & (pl.program_id(1) == 0))
    def _():
        acc_ref[...] = jnp.zeros_like(acc_ref)

    conv = _mean_conv(sum_ref[0], wpt_ref, bp_ref)
    acc_ref[0:1, :] += jnp.sum(conv, axis=0, keepdims=True)
    acc_ref[1:2, :] += jnp.sum(conv * conv, axis=0, keepdims=True)


def _tail2_body(sum_ref, wpt_ref, bp_ref, acc_ref, g_ref, bt_ref, out_ref):
    conv = _mean_conv(sum_ref[0], wpt_ref, bp_ref)
    n = float(B * HW)
    m = acc_ref[0:1, :] / n
    v = acc_ref[1:2, :] / n - m * m
    scale = g_ref[...] * jax.lax.rsqrt(v + 1e-5)
    shift = bt_ref[...] - m * scale
    y = jnp.maximum(conv * scale + shift, 0.0)
    out_ref[0] = y.T


def _tail(sums, Wp, bp, gamma, beta):
    wpt = Wp.T
    bp2 = bp.reshape(1, BEV_C)
    g2 = gamma.reshape(1, BEV_C)
    bt2 = beta.reshape(1, BEV_C)
    grid = (B, NB)
    sum_spec = pl.BlockSpec((1, CB, 128), lambda b, j: (b, j, 0))
    wpt_spec = pl.BlockSpec((EMB, BEV_C), lambda b, j: (0, 0))
    v_spec = pl.BlockSpec((1, BEV_C), lambda b, j: (0, 0))
    acc = pl.pallas_call(
        _tail1_body,
        grid=grid,
        in_specs=[sum_spec, wpt_spec, v_spec],
        out_specs=pl.BlockSpec((8, BEV_C), lambda b, j: (0, 0)),
        out_shape=jax.ShapeDtypeStruct((8, BEV_C), jnp.float32),
    )(sums, wpt, bp2)
    out = pl.pallas_call(
        _tail2_body,
        grid=grid,
        in_specs=[sum_spec, wpt_spec, v_spec,
                  pl.BlockSpec((8, BEV_C), lambda b, j: (0, 0)),
                  v_spec, v_spec],
        out_specs=pl.BlockSpec((1, BEV_C, CB), lambda b, j: (b, 0, j)),
        out_shape=jax.ShapeDtypeStruct((B, BEV_C, HW), jnp.float32),
    )(sums, wpt, bp2, acc, g2, bt2)
    return out.reshape(B, BEV_C, BEV_H, BEV_W)


def kernel(points, W1, b1, W2, b2, Wp, bp, gamma, beta):
    emb, ind = _mlp_embed(points, W1, b1, W2, b2)
    z2 = jnp.zeros((OB, 128), jnp.float32)
    sums = _sc_scatter(emb, ind, z2)
    return _tail(sums, Wp, bp, gamma, beta)


# speedup vs baseline: 3.0266x; 1.0004x over previous
import functools

import jax
import jax.numpy as jnp
from jax import lax
from jax.experimental import pallas as pl
from jax.experimental.pallas import tpu as pltpu
from jax.experimental.pallas import tpu_sc as plsc

B = 4
NP = 200000
PTS_IN = 4
EMB = 80
BEV_C = 128
BEV_H = 128
BEV_W = 128
HW = BEV_H * BEV_W
X_MIN, Y_MIN = -50.0, -50.0
X_MAX, Y_MAX = 50.0, 50.0
MX = (X_MAX - X_MIN) / BEV_W
MY = (Y_MAX - Y_MIN) / BEV_H

CA = 2000
NA = NP // CA
CB = 2048
NB = HW // CB


def _mlp_body(pts_ref, w1_ref, b1_ref, w2_ref, b2_ref, emb_ref, ind_ref):
    x = pts_ref[0]
    h = jnp.maximum(
        jnp.dot(x, w1_ref[...], preferred_element_type=jnp.float32) + b1_ref[...], 0.0)
    e = jnp.maximum(
        jnp.dot(h, w2_ref[...], preferred_element_type=jnp.float32) + b2_ref[...], 0.0)
    emb_ref[...] = jnp.concatenate(
        [e, jnp.ones((CA, 1), jnp.float32), jnp.zeros((CA, 127 - EMB), jnp.float32)],
        axis=1)
    xt = x.T
    xx = xt[0:1, :]
    yy = xt[1:2, :]
    ix = jnp.floor((xx - X_MIN) / MX).astype(jnp.int32)
    iy = jnp.floor((yy - Y_MIN) / MY).astype(jnp.int32)
    valid = (ix >= 0) & (ix < BEV_W) & (iy >= 0) & (iy < BEV_H)
    ind = jnp.clip(iy * BEV_W + ix, 0, HW - 1)
    ind_ref[0] = jnp.where(valid, ind, HW)


def _mlp_embed(points, W1, b1, W2, b2):
    grid = (B, NA)
    emb, ind = pl.pallas_call(
        _mlp_body,
        grid=grid,
        in_specs=[
            pl.BlockSpec((1, CA, PTS_IN), lambda b, j: (b, j, 0)),
            pl.BlockSpec((PTS_IN, EMB), lambda b, j: (0, 0)),
            pl.BlockSpec((1, EMB), lambda b, j: (0, 0)),
            pl.BlockSpec((EMB, EMB), lambda b, j: (0, 0)),
            pl.BlockSpec((1, EMB), lambda b, j: (0, 0)),
        ],
        out_specs=[
            pl.BlockSpec((CA, 128), lambda b, j: (b * NA + j, 0)),
            pl.BlockSpec((1, 1, CA), lambda b, j: (b * NA + j, 0, 0)),
        ],
        out_shape=[
            jax.ShapeDtypeStruct((B * NP, 128), jnp.float32),
            jax.ShapeDtypeStruct((B * NA, 1, CA), jnp.int32),
        ],
    )(points, W1, b1.reshape(1, EMB), W2, b2.reshape(1, EMB))
    return emb, ind.reshape(B * NP)


K = 128
NCH = NP // K
TAIL = NP - NCH * K
HALF = HW // 2
NROWS = 8320
ZR = NROWS // 16
OR_ = HALF // 16
OB = 128


def _sc_scatter_body(emb_hbm, ind_hbm, z2_hbm, sums_hbm,
                     acc, eb, ib, eb1, ib1, ebt, ibt, sem0, sem1):
    cid = lax.axis_index("c")
    sid = lax.axis_index("s")

    def remap(idx_ref, n, off):
        for i in range(n // 16):
            v = idx_ref[pl.ds(i * 16, 16)] - off
            bad = (v < 0) | (v >= HALF)
            idx_ref[pl.ds(i * 16, 16)] = jnp.where(bad, HALF, v)

    for rep in range(2):
        b = cid * 2 + rep
        for h in range(2):
            off = h * HALF
            pltpu.sync_copy(z2_hbm, eb)
            r0 = sid * ZR
            for j in range(4):
                pltpu.sync_copy(eb, acc.at[pl.ds(r0 + j * OB, OB)])
            pltpu.sync_copy(eb.at[pl.ds(0, 8)], acc.at[pl.ds(r0 + 4 * OB, 8)])
            plsc.subcore_barrier()

            def g_start(i, ebx, ibx, sem):
                c = i * 16 + sid

                @pl.when(c < NCH)
                def _():
                    base = b * NP + c * K
                    pltpu.async_copy(ind_hbm.at[pl.ds(base, K)], ibx, sem)
                    pltpu.async_copy(emb_hbm.at[pl.ds(base, K)], ebx, sem)

            def g_finish(i, ebx, ibx, sem):
                c = i * 16 + sid

                @pl.when(c < NCH)
                def _():
                    base = b * NP + c * K
                    pltpu.make_async_copy(
                        ind_hbm.at[pl.ds(base, K)], ibx, sem).wait()
                    pltpu.make_async_copy(
                        emb_hbm.at[pl.ds(base, K)], ebx, sem).wait()
                    remap(ibx, K, off)
                    pltpu.sync_copy(ebx, acc.at[ibx], add=True)

            def pair_body(t, _):
                g_start(2 * t + 1, eb1, ib1, sem1)
                g_finish(2 * t, eb, ib, sem0)
                g_start(2 * t + 2, eb, ib, sem0)
                g_finish(2 * t + 1, eb1, ib1, sem1)
                return ()

            g_start(0, eb, ib, sem0)
            lax.fori_loop(0, (NCH + 15) // 16 // 2, pair_body, (),
                          unroll=False)

            @pl.when(sid == NCH % 16)
            def _():
                base = b * NP + NCH * K
                pltpu.sync_copy(ind_hbm.at[pl.ds(base, TAIL)], ibt)
                pltpu.sync_copy(emb_hbm.at[pl.ds(base, TAIL)], ebt)
                remap(ibt, TAIL, off)
                pltpu.sync_copy(ebt, acc.at[ibt], add=True)

            plsc.subcore_barrier()

            o0 = sid * OR_
            for j in range(OR_ // OB):
                pltpu.sync_copy(acc.at[pl.ds(o0 + j * OB, OB)], eb)
                pltpu.sync_copy(eb, sums_hbm.at[b, pl.ds(off + o0 + j * OB, OB)])
            plsc.subcore_barrier()


@functools.partial(
    pl.kernel,
    out_type=jax.ShapeDtypeStruct((B, HW, 128), jnp.float32),
    mesh=plsc.VectorSubcoreMesh(core_axis_name="c", subcore_axis_name="s",
                                num_cores=2, num_subcores=16),
    scratch_types=[
        pltpu.VMEM_SHARED((NROWS, 128), jnp.float32),
        pltpu.VMEM((K, 128), jnp.float32),
        pltpu.VMEM((K,), jnp.int32),
        pltpu.VMEM((K, 128), jnp.float32),
        pltpu.VMEM((K,), jnp.int32),
        pltpu.VMEM((TAIL, 128), jnp.float32),
        pltpu.VMEM((TAIL,), jnp.int32),
        pltpu.SemaphoreType.DMA,
        pltpu.SemaphoreType.DMA,
    ],
)
def _sc_scatter(emb_hbm, ind_hbm, z2_hbm, sums_hbm,
                acc, eb, ib, eb1, ib1, ebt, ibt, sem0, sem1):
    _sc_scatter_body(emb_hbm, ind_hbm, z2_hbm, sums_hbm,
                     acc, eb, ib, eb1, ib1, ebt, ibt, sem0, sem1)


def _mean_conv(blk, wpt_ref, bp_ref):
    s = blk[:, :EMB]
    c = blk[:, EMB:EMB + 1]
    recip = 1.0 / jnp.maximum(c, 1.0)
    mean = s * recip
    return jnp.dot(mean, wpt_ref[...],
                   preferred_element_type=jnp.float32) + bp_ref[...]


def _tail1_body(sum_ref, wpt_ref, bp_ref, acc_ref):
    @pl.when((pl.program_id(0) == 0) & (pl.program_id(1) == 0))
    def _():
        acc_ref[...] = jnp.zeros_like(acc_ref)

    conv = _mean_conv(sum_ref[0], wpt_ref, bp_ref)
    acc_ref[0:1, :] += jnp.sum(conv, axis=0, keepdims=True)
    acc_ref[1:2, :] += jnp.sum(conv * conv, axis=0, keepdims=True)


def _tail2_body(sum_ref, wpt_ref, bp_ref, acc_ref, g_ref, bt_ref, out_ref):
    conv = _mean_conv(sum_ref[0], wpt_ref, bp_ref)
    n = float(B * HW)
    m = acc_ref[0:1, :] / n
    v = acc_ref[1:2, :] / n - m * m
    scale = g_ref[...] * jax.lax.rsqrt(v + 1e-5)
    shift = bt_ref[...] - m * scale
    y = jnp.maximum(conv * scale + shift, 0.0)
    out_ref[0] = y.T


def _tail(sums, Wp, bp, gamma, beta):
    wpt = Wp.T
    bp2 = bp.reshape(1, BEV_C)
    g2 = gamma.reshape(1, BEV_C)
    bt2 = beta.reshape(1, BEV_C)
    grid = (B, NB)
    sum_spec = pl.BlockSpec((1, CB, 128), lambda b, j: (b, j, 0))
    wpt_spec = pl.BlockSpec((EMB, BEV_C), lambda b, j: (0, 0))
    v_spec = pl.BlockSpec((1, BEV_C), lambda b, j: (0, 0))
    acc = pl.pallas_call(
        _tail1_body,
        grid=grid,
        in_specs=[sum_spec, wpt_spec, v_spec],
        out_specs=pl.BlockSpec((8, BEV_C), lambda b, j: (0, 0)),
        out_shape=jax.ShapeDtypeStruct((8, BEV_C), jnp.float32),
    )(sums, wpt, bp2)
    out = pl.pallas_call(
        _tail2_body,
        grid=grid,
        in_specs=[sum_spec, wpt_spec, v_spec,
                  pl.BlockSpec((8, BEV_C), lambda b, j: (0, 0)),
                  v_spec, v_spec],
        out_specs=pl.BlockSpec((1, BEV_C, CB), lambda b, j: (b, 0, j)),
        out_shape=jax.ShapeDtypeStruct((B, BEV_C, HW), jnp.float32),
    )(sums, wpt, bp2, acc, g2, bt2)
    return out.reshape(B, BEV_C, BEV_H, BEV_W)


def kernel(points, W1, b1, W2, b2, Wp, bp, gamma, beta):
    emb, ind = _mlp_embed(points, W1, b1, W2, b2)
    z2 = jnp.zeros((OB, 128), jnp.float32)
    sums = _sc_scatter(emb, ind, z2)
    return _tail(sums, Wp, bp, gamma, beta)
